# baseline (speedup 1.0000x reference)
import jax
import jax.numpy as jnp
from jax import lax
from jax.experimental import pallas as pl
from jax.experimental.pallas import tpu as pltpu
from jax.experimental.pallas import tpu_sc as plsc

BATCH = 4096
SEQ_LEN = 200
EMBED_DIM = 32
B_TOTAL = BATCH * SEQ_LEN
NUM_WORKERS = 32
CHUNK = 512
NCH_W = (B_TOTAL // CHUNK) // NUM_WORKERS


def _gather_pack_kernel(table_hbm, idx_hbm, out_hbm, idx_v, rows_v, pk_v,
                        sem_i0, sem_i1, sem_g0, sem_g1, sem_o0, sem_o1):
    sems_i = (sem_i0, sem_i1)
    sems_g = (sem_g0, sem_g1)
    sems_o = (sem_o0, sem_o1)

    wid = lax.axis_index("s") * 2 + lax.axis_index("c")
    c0 = wid * NCH_W
    clast = c0 + NCH_W - 1

    iota = lax.iota(jnp.int32, 16)

    def start_idx(c, b):
        pltpu.async_copy(idx_hbm.at[pl.ds(c * CHUNK, CHUNK)], idx_v.at[b],
                         sems_i[b])

    def wait_idx(c, b):
        pltpu.make_async_copy(idx_hbm.at[pl.ds(c * CHUNK, CHUNK)],
                              idx_v.at[b], sems_i[b]).wait()

    def start_gather(b):
        pltpu.async_copy(table_hbm.at[idx_v.at[b]], rows_v.at[b], sems_g[b])

    def wait_gather(b):
        pltpu.make_async_copy(table_hbm.at[idx_v.at[b]], rows_v.at[b],
                              sems_g[b]).wait()

    rv_lo = (iota // 8) * 8 + (iota % 8)
    rv_hi = rv_lo + 16

    def pack(b):
        for k in range(4):
            rlo = rv_lo + k * 32
            rhi = rv_hi + k * 32

            @plsc.parallel_loop(0, 128, unroll=8)
            def _col(bl):
                col = jnp.full((16,), 0, jnp.int32) + bl
                x0 = rows_v[b, k * 128 + bl, pl.ds(0, 16)]
                x1 = rows_v[b, k * 128 + bl, pl.ds(16, 16)]
                plsc.store_scatter(pk_v.at[b], [rlo, col], x0)
                plsc.store_scatter(pk_v.at[b], [rhi, col], x1)

    def _wb_pairs(c, b):
        l = c // 8
        bg0 = (c % 8) * 4
        for dg in range(4):
            r0 = ((l * 4 + dg) * 32 + bg0) * 8
            for k in range(4):
                src = pk_v.at[b, pl.ds(k * 32 + dg * 8, 8), pl.ds(0, 128)]
                dst = out_hbm.at[pl.ds(r0 + k * 8, 8), pl.ds(0, 128)]
                yield src, dst

    def start_wb(c, b):
        for src, dst in _wb_pairs(c, b):
            pltpu.async_copy(src, dst, sems_o[b])

    def wait_wb(c, b):
        for src, dst in _wb_pairs(c, b):
            pltpu.make_async_copy(src, dst, sems_o[b]).wait()

    start_idx(c0, 0)
    start_idx(c0 + 1, 1)
    wait_idx(c0, 0)
    start_gather(0)
    wait_gather(0)
    wait_idx(c0 + 1, 1)
    start_gather(1)
    pack(0)
    start_idx(c0 + 2, 0)
    start_wb(c0, 0)

    def body(i, b):
        c = c0 + i
        wait_gather(b)
        wait_idx(c + 1, 1 - b)
        wait_wb(c - 1, 1 - b)
        start_gather(1 - b)
        pack(b)
        start_idx(jnp.minimum(c + 2, clast), b)
        start_wb(c, b)

    def pair(p, carry):
        body(2 * p + 1, 1)
        body(2 * p + 2, 0)
        return carry

    lax.fori_loop(0, (NCH_W - 2) // 2, pair, 0)

    wait_gather(1)
    pack(1)
    start_wb(clast, 1)

    wait_wb(clast - 1, 0)
    wait_wb(clast, 1)
    wait_idx(clast, 0)


NVG = 7812
NVG_W_MAX = 245


def _detile_kernel(tabT_hbm, out_hbm, tb_v, pkt_v, sem_in, sem_out):
    wid = lax.axis_index("s") * 2 + lax.axis_index("c")
    iota = lax.iota(jnp.int32, 16)
    dg_lo = iota // 8
    dg_hi = dg_lo + 2
    dl_v = iota % 8

    def do_block(vg):
        for dg in range(4):
            pltpu.async_copy(
                tabT_hbm.at[pl.ds(dg * 8, 8), pl.ds(vg * 128, 128)],
                tb_v.at[dg, pl.ds(0, 8), pl.ds(0, 128)], sem_in)
        for dg in range(4):
            pltpu.make_async_copy(
                tabT_hbm.at[pl.ds(dg * 8, 8), pl.ds(vg * 128, 128)],
                tb_v.at[dg, pl.ds(0, 8), pl.ds(0, 128)], sem_in).wait()

        @plsc.parallel_loop(0, 128, unroll=8)
        def _col(vloc):
            col = jnp.full((16,), 0, jnp.int32) + vloc
            x0 = plsc.load_gather(tb_v, [dg_lo, dl_v, col])
            x1 = plsc.load_gather(tb_v, [dg_hi, dl_v, col])
            r = vloc // 4
            c32 = (vloc - r * 4) * 32
            pkt_v[r, pl.ds(c32, 16)] = x0
            pkt_v[r, pl.ds(c32 + 16, 16)] = x1

        pltpu.async_copy(pkt_v, out_hbm.at[pl.ds(vg * 32, 32)], sem_out)
        pltpu.make_async_copy(pkt_v, out_hbm.at[pl.ds(vg * 32, 32)],
                              sem_out).wait()

    def step(i, carry):
        vg = wid + i * 32

        @pl.when(vg < NVG)
        def _():
            do_block(vg)
        return carry

    lax.fori_loop(0, NVG_W_MAX, step, 0)


@jax.jit
def kernel(seq, emb_table):
    flat_idx = seq.T.reshape(B_TOTAL)

    detile = pl.kernel(
        _detile_kernel,
        out_type=jax.ShapeDtypeStruct((250000, 128), jnp.float32),
        mesh=plsc.VectorSubcoreMesh(core_axis_name="c", subcore_axis_name="s"),
        scratch_types=[
            pltpu.VMEM((4, 8, 129), jnp.float32),
            pltpu.VMEM((32, 128), jnp.float32),
            pltpu.SemaphoreType.DMA,
            pltpu.SemaphoreType.DMA,
        ],
        compiler_params=pltpu.CompilerParams(
            use_tc_tiling_on_sc=True, needs_layout_passes=False),
    )
    table_1d = detile(emb_table.T).reshape(1000000 * EMBED_DIM)
    tail_flat = emb_table[999936:].reshape(64 * EMBED_DIM)
    table_1d = lax.dynamic_update_slice(table_1d, tail_flat,
                                        (999936 * EMBED_DIM,))
    table_lin = table_1d.reshape(1000000, EMBED_DIM)

    call = pl.kernel(
        _gather_pack_kernel,
        out_type=jax.ShapeDtypeStruct((B_TOTAL * EMBED_DIM // 128, 128),
                                      jnp.float32),
        mesh=plsc.VectorSubcoreMesh(core_axis_name="c", subcore_axis_name="s"),
        scratch_types=[
            pltpu.VMEM((2, CHUNK), jnp.int32),
            pltpu.VMEM((2, CHUNK, EMBED_DIM), jnp.float32),
            pltpu.VMEM((2, 128, 129), jnp.float32),
        ] + [pltpu.SemaphoreType.DMA] * 6,
        compiler_params=pltpu.CompilerParams(
            use_tc_tiling_on_sc=False, needs_layout_passes=False),
    )
    out = call(table_lin, flat_idx)
    out5 = out.reshape(SEQ_LEN, 4, 32, 8, 128)
    return out5.transpose(2, 4, 0, 1, 3).reshape(BATCH, SEQ_LEN, EMBED_DIM)


# speedup vs baseline: 1.4556x; 1.4556x over previous
import jax
import jax.numpy as jnp
from jax import lax
from jax.experimental import pallas as pl
from jax.experimental.pallas import tpu as pltpu
from jax.experimental.pallas import tpu_sc as plsc

BATCH = 4096
SEQ_LEN = 200
EMBED_DIM = 32
B_TOTAL = BATCH * SEQ_LEN
NUM_WORKERS = 32
CHUNK = 512
NCH_W = (B_TOTAL // CHUNK) // NUM_WORKERS


def _gather_pack_kernel(table_hbm, idx_hbm, out_hbm, idx_v, rows_v, pk_v,
                        sem_i0, sem_i1, sem_g0, sem_g1, sem_o0, sem_o1):
    sems_i = (sem_i0, sem_i1)
    sems_g = (sem_g0, sem_g1)
    sems_o = (sem_o0, sem_o1)

    wid = lax.axis_index("s") * 2 + lax.axis_index("c")
    c0 = wid * NCH_W
    clast = c0 + NCH_W - 1

    iota = lax.iota(jnp.int32, 16)

    def start_idx(c, b):
        pltpu.async_copy(idx_hbm.at[pl.ds(c * CHUNK, CHUNK)], idx_v.at[b],
                         sems_i[b])

    def wait_idx(c, b):
        pltpu.make_async_copy(idx_hbm.at[pl.ds(c * CHUNK, CHUNK)],
                              idx_v.at[b], sems_i[b]).wait()

    def start_gather(b):
        pltpu.async_copy(table_hbm.at[idx_v.at[b]], rows_v.at[b], sems_g[b])

    def wait_gather(b):
        pltpu.make_async_copy(table_hbm.at[idx_v.at[b]], rows_v.at[b],
                              sems_g[b]).wait()

    rv_lo = (iota // 8) * 8 + (iota % 8)
    rv_hi = rv_lo + 16

    def pack(b):
        for k in range(4):
            rlo = rv_lo + k * 32
            rhi = rv_hi + k * 32

            @plsc.parallel_loop(0, 128, unroll=8)
            def _col(bl):
                col = jnp.full((16,), 0, jnp.int32) + bl
                x0 = rows_v[b, k * 128 + bl, pl.ds(0, 16)]
                x1 = rows_v[b, k * 128 + bl, pl.ds(16, 16)]
                plsc.store_scatter(pk_v.at[b], [rlo, col], x0)
                plsc.store_scatter(pk_v.at[b], [rhi, col], x1)

    def _wb_pairs(c, b):
        l = c // 8
        bg0 = (c % 8) * 4
        for dg in range(4):
            r0 = ((l * 4 + dg) * 32 + bg0) * 8
            for k in range(4):
                src = pk_v.at[b, pl.ds(k * 32 + dg * 8, 8), pl.ds(0, 128)]
                dst = out_hbm.at[pl.ds(r0 + k * 8, 8), pl.ds(0, 128)]
                yield src, dst

    def start_wb(c, b):
        for src, dst in _wb_pairs(c, b):
            pltpu.async_copy(src, dst, sems_o[b])

    def wait_wb(c, b):
        for src, dst in _wb_pairs(c, b):
            pltpu.make_async_copy(src, dst, sems_o[b]).wait()

    start_idx(c0, 0)
    start_idx(c0 + 1, 1)
    wait_idx(c0, 0)
    start_gather(0)
    wait_gather(0)
    wait_idx(c0 + 1, 1)
    start_gather(1)
    pack(0)
    start_idx(c0 + 2, 0)
    start_wb(c0, 0)

    def body(i, b):
        c = c0 + i
        wait_gather(b)
        wait_idx(c + 1, 1 - b)
        wait_wb(c - 1, 1 - b)
        start_gather(1 - b)
        pack(b)
        start_idx(jnp.minimum(c + 2, clast), b)
        start_wb(c, b)

    def pair(p, carry):
        body(2 * p + 1, 1)
        body(2 * p + 2, 0)
        return carry

    lax.fori_loop(0, (NCH_W - 2) // 2, pair, 0)

    wait_gather(1)
    pack(1)
    start_wb(clast, 1)

    wait_wb(clast - 1, 0)
    wait_wb(clast, 1)
    wait_idx(clast, 0)


NVG = 7812
NVG_W_MAX = 245


def _detile_kernel(tabT_hbm, out_hbm, tb_v, pkt_v,
                   sem_i0, sem_i1, sem_o0, sem_o1):
    wid = lax.axis_index("s") * 2 + lax.axis_index("c")
    iota = lax.iota(jnp.int32, 16)
    dg_lo = iota // 8
    dg_hi = dg_lo + 2
    dl_v = iota % 8
    sems_i = (sem_i0, sem_i1)
    sems_o = (sem_o0, sem_o1)

    def vg_of(i):
        return wid + i * 32

    def start_in(i, b):
        for dg in range(4):
            pltpu.async_copy(
                tabT_hbm.at[pl.ds(dg * 8, 8), pl.ds(vg_of(i) * 128, 128)],
                tb_v.at[b, dg, pl.ds(0, 8), pl.ds(0, 128)], sems_i[b])

    def wait_in(i, b):
        for dg in range(4):
            pltpu.make_async_copy(
                tabT_hbm.at[pl.ds(dg * 8, 8), pl.ds(vg_of(i) * 128, 128)],
                tb_v.at[b, dg, pl.ds(0, 8), pl.ds(0, 128)], sems_i[b]).wait()

    def pack(b):
        @plsc.parallel_loop(0, 128, unroll=8)
        def _col(vloc):
            col = jnp.full((16,), 0, jnp.int32) + vloc
            x0 = plsc.load_gather(tb_v.at[b], [dg_lo, dl_v, col])
            x1 = plsc.load_gather(tb_v.at[b], [dg_hi, dl_v, col])
            r = vloc // 4
            c32 = (vloc - r * 4) * 32
            pkt_v[b, r, pl.ds(c32, 16)] = x0
            pkt_v[b, r, pl.ds(c32 + 16, 16)] = x1

    def start_out(i, b):
        pltpu.async_copy(pkt_v.at[b], out_hbm.at[pl.ds(vg_of(i) * 32, 32)],
                         sems_o[b])

    def wait_out(i, b):
        pltpu.make_async_copy(pkt_v.at[b],
                              out_hbm.at[pl.ds(vg_of(i) * 32, 32)],
                              sems_o[b]).wait()

    NB = NVG // 32

    start_in(0, 0)
    wait_in(0, 0)
    start_in(1, 1)
    pack(0)
    start_out(0, 0)
    wait_in(1, 1)
    start_in(2, 0)
    pack(1)
    start_out(1, 1)
    wait_in(2, 0)
    start_in(3, 1)
    wait_out(0, 0)
    pack(0)
    start_out(2, 0)

    def spair(p, carry):
        for (off, b) in ((3, 1), (4, 0)):
            i = 2 * p + off
            wait_in(i, b)
            start_in(i + 1, 1 - b)
            wait_out(i - 2, b)
            pack(b)
            start_out(i, b)
        return carry

    lax.fori_loop(0, (NB - 4) // 2, spair, 0)

    wait_in(NB - 1, 1)
    wait_out(NB - 3, 1)
    pack(1)
    start_out(NB - 1, 1)
    wait_out(NB - 2, 0)
    wait_out(NB - 1, 1)

    @pl.when(wid < 4)
    def _leftover():
        start_in(NB, 0)
        wait_in(NB, 0)
        pack(0)
        start_out(NB, 0)
        wait_out(NB, 0)


@jax.jit
def kernel(seq, emb_table):
    flat_idx = seq.T.reshape(B_TOTAL)

    detile = pl.kernel(
        _detile_kernel,
        out_type=jax.ShapeDtypeStruct((250000, 128), jnp.float32),
        mesh=plsc.VectorSubcoreMesh(core_axis_name="c", subcore_axis_name="s"),
        scratch_types=[
            pltpu.VMEM((2, 4, 8, 129), jnp.float32),
            pltpu.VMEM((2, 32, 128), jnp.float32),
        ] + [pltpu.SemaphoreType.DMA] * 4,
        compiler_params=pltpu.CompilerParams(
            use_tc_tiling_on_sc=True, needs_layout_passes=False),
    )
    table_1d = detile(emb_table.T).reshape(1000000 * EMBED_DIM)
    tail_flat = emb_table[999936:].reshape(64 * EMBED_DIM)
    table_1d = lax.dynamic_update_slice(table_1d, tail_flat,
                                        (999936 * EMBED_DIM,))
    table_lin = table_1d.reshape(1000000, EMBED_DIM)

    call = pl.kernel(
        _gather_pack_kernel,
        out_type=jax.ShapeDtypeStruct((B_TOTAL * EMBED_DIM // 128, 128),
                                      jnp.float32),
        mesh=plsc.VectorSubcoreMesh(core_axis_name="c", subcore_axis_name="s"),
        scratch_types=[
            pltpu.VMEM((2, CHUNK), jnp.int32),
            pltpu.VMEM((2, CHUNK, EMBED_DIM), jnp.float32),
            pltpu.VMEM((2, 128, 129), jnp.float32),
        ] + [pltpu.SemaphoreType.DMA] * 6,
        compiler_params=pltpu.CompilerParams(
            use_tc_tiling_on_sc=False, needs_layout_passes=False),
    )
    out = call(table_lin, flat_idx)
    out5 = out.reshape(SEQ_LEN, 4, 32, 8, 128)
    return out5.transpose(2, 4, 0, 1, 3).reshape(BATCH, SEQ_LEN, EMBED_DIM)
